# Initial kernel scaffold; baseline (speedup 1.0000x reference)
import jax
import jax.numpy as jnp
from jax.experimental import pallas as pl


def kernel(x, table):
    raise NotImplementedError("write your pallas kernel here")


# speedup vs baseline: 1.6843x; 1.6843x over previous
import functools

import jax
import jax.numpy as jnp
from jax import lax
from jax.experimental import pallas as pl
from jax.experimental.pallas import tpu as pltpu
from jax.experimental.pallas import tpu_sc as plsc

CHUNK = 128


@functools.lru_cache(maxsize=None)
def _build(n_rows: int, d_embed: int):
    info = plsc.get_sparse_core_info()
    nw = info.num_cores * info.num_subcores
    per_w = n_rows // nw
    n_chunks = per_w // CHUNK
    nc = info.num_cores

    mesh = plsc.VectorSubcoreMesh(core_axis_name="c", subcore_axis_name="s")

    @functools.partial(
        pl.kernel,
        mesh=mesh,
        out_type=jax.ShapeDtypeStruct((n_rows, d_embed), jnp.float32),
        scratch_types=[
            pltpu.VMEM((n_chunks, CHUNK), jnp.int32),
            pltpu.VMEM((CHUNK, d_embed), jnp.float32),
            pltpu.SemaphoreType.DMA,
        ],
        compiler_params=pltpu.CompilerParams(use_tc_tiling_on_sc=False),
    )
    def k(x_hbm, table_hbm, out_hbm, idx_v, rows_v, sem):
        wid = lax.axis_index("s") * nc + lax.axis_index("c")
        row_base = wid * n_chunks
        out_base = wid * per_w
        pltpu.sync_copy(x_hbm.at[pl.ds(row_base, n_chunks)], idx_v)

        def body(c, carry):
            pltpu.async_copy(table_hbm.at[idx_v.at[c]], rows_v, sem).wait()
            pltpu.sync_copy(
                rows_v, out_hbm.at[pl.ds(out_base + c * CHUNK, CHUNK)]
            )
            return carry

        lax.fori_loop(0, n_chunks, body, 0)

    return k


def kernel(x, table):
    orig_shape = x.shape
    d_embed = table.shape[1]
    flat = x.reshape(-1).astype(jnp.int32)
    n = flat.shape[0]
    group = 32 * CHUNK
    n_pad = -n % group
    if n_pad:
        flat = jnp.concatenate([flat, jnp.zeros((n_pad,), jnp.int32)])
    x2d = flat.reshape(-1, CHUNK)
    out = _build(flat.shape[0], d_embed)(x2d, table)
    if n_pad:
        out = out[:n]
    return out.reshape(*orig_shape, d_embed)


# speedup vs baseline: 1.8790x; 1.1156x over previous
import functools

import jax
import jax.numpy as jnp
from jax import lax
from jax.experimental import pallas as pl
from jax.experimental.pallas import tpu as pltpu
from jax.experimental.pallas import tpu_sc as plsc

CHUNK = 128
NBUF = 8
LAG = 4


@functools.lru_cache(maxsize=None)
def _build(n_rows: int, d_embed: int):
    info = plsc.get_sparse_core_info()
    nw = info.num_cores * info.num_subcores
    per_w = n_rows // nw
    n_chunks = per_w // CHUNK
    nc = info.num_cores

    mesh = plsc.VectorSubcoreMesh(core_axis_name="c", subcore_axis_name="s")

    @functools.partial(
        pl.kernel,
        mesh=mesh,
        out_type=jax.ShapeDtypeStruct((n_rows, d_embed), jnp.float32),
        scratch_types=[
            pltpu.VMEM((n_chunks, CHUNK), jnp.int32),
            pltpu.VMEM((NBUF, CHUNK, d_embed), jnp.float32),
            pltpu.SemaphoreType.DMA,
            pltpu.SemaphoreType.DMA,
        ],
        compiler_params=pltpu.CompilerParams(use_tc_tiling_on_sc=False),
    )
    def k(x_hbm, table_hbm, out_hbm, idx_v, rows_v, gsem, osem):
        wid = lax.axis_index("s") * nc + lax.axis_index("c")
        row_base = wid * n_chunks
        out_base = wid * per_w
        pltpu.sync_copy(x_hbm.at[pl.ds(row_base, n_chunks)], idx_v)

        def gather(c, b):
            return pltpu.make_async_copy(
                table_hbm.at[idx_v.at[c]], rows_v.at[b], gsem
            )

        def writeback(c, b):
            return pltpu.make_async_copy(
                rows_v.at[b], out_hbm.at[pl.ds(out_base + c * CHUNK, CHUNK)], osem
            )

        def body(c, carry):
            b = lax.rem(c, NBUF)

            @pl.when(c >= NBUF)
            def _():
                writeback(c - NBUF, b).wait()

            gather(c, b).start()

            @pl.when(c >= LAG)
            def _():
                cw = c - LAG
                bw = lax.rem(cw, NBUF)
                gather(cw, bw).wait()
                writeback(cw, bw).start()

            return carry

        lax.fori_loop(0, n_chunks, body, 0)
        for cw in range(n_chunks - LAG, n_chunks):
            gather(cw, cw % NBUF).wait()
            writeback(cw, cw % NBUF).start()
        for cw in range(n_chunks - NBUF, n_chunks):
            writeback(cw, cw % NBUF).wait()

    return k


def kernel(x, table):
    orig_shape = x.shape
    d_embed = table.shape[1]
    flat = x.reshape(-1).astype(jnp.int32)
    n = flat.shape[0]
    group = 32 * CHUNK
    n_pad = -n % group
    if n_pad:
        flat = jnp.concatenate([flat, jnp.zeros((n_pad,), jnp.int32)])
    x2d = flat.reshape(-1, CHUNK)
    out = _build(flat.shape[0], d_embed)(x2d, table)
    if n_pad:
        out = out[:n]
    return out.reshape(*orig_shape, d_embed)
